# baseline (speedup 1.0000x reference)
import functools

import jax
import jax.numpy as jnp
from jax import lax
from jax.experimental import pallas as pl
from jax.experimental.pallas import tpu as pltpu
from jax.experimental.pallas import tpu_sc as plsc

N = 10000
D = 128
E = 320000
BN_EPS = 1e-5

NC, NS, L = 2, 16, 16
NW = NC * NS
EPW = E // NW
K = 80
NCHUNK = EPW // K
RPT = 624
TAIL = N - RPT * NS


def _sc_segment(h, packed):
    mesh = plsc.VectorSubcoreMesh(core_axis_name="c", subcore_axis_name="s")

    NR = 4

    @functools.partial(
        pl.kernel,
        out_type=jax.ShapeDtypeStruct((NC, N, D), jnp.float32),
        mesh=mesh,
        scratch_types=(
            [pltpu.VMEM((K, D), jnp.float32) for _ in range(NR)]
            + [pltpu.VMEM((3, K), jnp.int32) for _ in range(NR)]
            + [pltpu.VMEM_SHARED((N, D), jnp.float32)]
            + [pltpu.SemaphoreType.DMA for _ in range(2 * NR)]
        ),
    )
    def seg(h_hbm, pk_hbm, out_hbm, *scr):
        rows = scr[:NR]
        desc = scr[NR:2 * NR]
        acc = scr[2 * NR]
        gsem = scr[2 * NR + 1:3 * NR + 1]
        ssem = scr[3 * NR + 1:]
        c = lax.axis_index("c")
        s = lax.axis_index("s")
        wid = s * NC + c
        base = wid * NCHUNK

        def gather(j, b):
            pltpu.sync_copy(pk_hbm.at[base + j], desc[b])
            pltpu.async_copy(h_hbm.at[desc[b].at[0]], rows[b], gsem[b])

        def wait_scatter(b):
            pltpu.make_async_copy(rows[b], acc.at[desc[b].at[1]],
                                  ssem[b]).wait()

        def wait_gather(b):
            pltpu.make_async_copy(h_hbm.at[desc[b].at[0]], rows[b],
                                  gsem[b]).wait()

        def process(b):
            def scale(t, carry2):
                m16 = desc[b][2, pl.ds(t * L, L)]
                for e in range(L):
                    m = lax.bitcast_convert_type(m16[e], jnp.float32)
                    r = t * L + e
                    for g in range(D // L):
                        rows[b][r, pl.ds(g * L, L)] = (
                            rows[b][r, pl.ds(g * L, L)] * m)
                return carry2

            lax.fori_loop(0, K // L, scale, 0)
            pltpu.async_copy(rows[b], acc.at[desc[b].at[1]], ssem[b],
                             add=True)

        def slot(j, t, first=False, g_ok=True):
            wait_gather(t % NR)
            if not first:
                wait_scatter((t - 1) % NR)
            if g_ok:
                gather(j + NR - 1, (t + NR - 1) % NR)
            process(t % NR)

        gather(0, 0)
        gather(1, 1)
        gather(2, 2)

        pltpu.sync_copy(h_hbm.at[pl.ds(s * RPT, RPT)], acc.at[pl.ds(s * RPT, RPT)])

        @pl.when(s == NS - 1)
        def _():
            pltpu.sync_copy(h_hbm.at[pl.ds(RPT * NS, TAIL)],
                            acc.at[pl.ds(RPT * NS, TAIL)])

        plsc.subcore_barrier()

        def body3(i, carry):
            j0 = NR * i
            for t in range(NR):
                slot(j0 + t, t, first=False)
            return carry

        M = (NCHUNK - NR - (NR - 2)) // NR
        for t in range(NR):
            slot(t, t, first=(t == 0))
        lax.fori_loop(1, 1 + M, body3, 0)
        for j in range(NR * (1 + M), NCHUNK):
            slot(j, j % NR, g_ok=(j + NR - 1 < NCHUNK))
        wait_scatter((NCHUNK - 1) % NR)

        plsc.subcore_barrier()
        pltpu.sync_copy(acc.at[pl.ds(s * RPT, RPT)],
                        out_hbm.at[c, pl.ds(s * RPT, RPT)])

        @pl.when(s == NS - 1)
        def _():
            pltpu.sync_copy(acc.at[pl.ds(RPT * NS, TAIL)],
                            out_hbm.at[c, pl.ds(RPT * NS, TAIL)])

    return seg(h, packed)


def _tc_tail(h, p0, p1, snorm_n, W1, b1, W2, b2, gamma, beta):
    def body(h_ref, p0_ref, p1_ref, sn_ref, w1_ref, b1_ref, w2_ref, b2_ref,
             g_ref, be_ref, o_ref):
        hh = h_ref[...]
        x = p0_ref[...] + p1_ref[...] - hh
        x = jnp.maximum(
            jnp.dot(x, w1_ref[...], preferred_element_type=jnp.float32)
            + b1_ref[...], 0.0)
        x = jnp.dot(x, w2_ref[...], preferred_element_type=jnp.float32) + b2_ref[...]
        x = x * sn_ref[...]
        mean = jnp.mean(x, axis=0, keepdims=True)
        xc = x - mean
        var = jnp.mean(xc * xc, axis=0, keepdims=True)
        y = xc * lax.rsqrt(var + BN_EPS) * g_ref[...] + be_ref[...]
        o_ref[...] = hh + jnp.maximum(y, 0.0)

    return pl.pallas_call(
        body,
        out_shape=jax.ShapeDtypeStruct((N, D), jnp.float32),
    )(h, p0, p1, snorm_n, W1, b1, W2, b2, gamma, beta)


def kernel(h, edge_index, edge_mask, snorm_n, W1, b1, W2, b2, gamma, beta):
    src = edge_index[0].reshape(E // K, K)
    dst = edge_index[1].reshape(E // K, K)
    mbits = lax.bitcast_convert_type(edge_mask[:, 0], jnp.int32).reshape(E // K, K)
    packed = jnp.stack([src, dst, mbits], axis=1)
    part = _sc_segment(h, packed)
    return _tc_tail(h, part[0], part[1], snorm_n, W1, b1, W2, b2, gamma, beta)


# speedup vs baseline: 1.0629x; 1.0629x over previous
import functools

import jax
import jax.numpy as jnp
from jax import lax
from jax.experimental import pallas as pl
from jax.experimental.pallas import tpu as pltpu
from jax.experimental.pallas import tpu_sc as plsc

N = 10000
D = 128
E = 320000
BN_EPS = 1e-5

NC, NS, L = 2, 16, 16
NW = NC * NS
EPW = E // NW
K = 80
NCHUNK = EPW // K
RPT = 624
TAIL = N - RPT * NS


def _sc_segment(h, packed):
    mesh = plsc.VectorSubcoreMesh(core_axis_name="c", subcore_axis_name="s")

    NR = 3

    @functools.partial(
        pl.kernel,
        out_type=jax.ShapeDtypeStruct((NC, N, D), jnp.float32),
        mesh=mesh,
        scratch_types=(
            [pltpu.VMEM((K, D), jnp.float32) for _ in range(NR)]
            + [pltpu.VMEM((3, K), jnp.int32) for _ in range(NR)]
            + [pltpu.VMEM_SHARED((N, D), jnp.float32)]
            + [pltpu.SemaphoreType.DMA for _ in range(2 * NR)]
        ),
    )
    def seg(h_hbm, pk_hbm, out_hbm, *scr):
        rows = scr[:NR]
        desc = scr[NR:2 * NR]
        acc = scr[2 * NR]
        gsem = scr[2 * NR + 1:3 * NR + 1]
        ssem = scr[3 * NR + 1:]
        c = lax.axis_index("c")
        s = lax.axis_index("s")
        wid = s * NC + c
        base = wid * NCHUNK

        def gather(j, b):
            pltpu.sync_copy(pk_hbm.at[base + j], desc[b])
            pltpu.async_copy(h_hbm.at[desc[b].at[0]], rows[b], gsem[b])

        def wait_scatter(b):
            pltpu.make_async_copy(rows[b], acc.at[desc[b].at[1]],
                                  ssem[b]).wait()

        def process(b, mid):
            pltpu.make_async_copy(h_hbm.at[desc[b].at[0]], rows[b],
                                  gsem[b]).wait()

            def scale(t, carry2):
                m16 = desc[b][2, pl.ds(t * L, L)]
                for e in range(L):
                    m = lax.bitcast_convert_type(m16[e], jnp.float32)
                    r = t * L + e
                    for g in range(D // L):
                        rows[b][r, pl.ds(g * L, L)] = (
                            rows[b][r, pl.ds(g * L, L)] * m)
                return carry2

            lax.fori_loop(0, 2, scale, 0)
            mid()
            lax.fori_loop(2, K // L, scale, 0)
            pltpu.async_copy(rows[b], acc.at[desc[b].at[1]], ssem[b],
                             add=True)

        def slot(j, t, first=False, g_ok=True):
            def mid():
                if not first:
                    wait_scatter((t - 1) % NR)
                if g_ok:
                    gather(j + 2, (t + 2) % NR)

            process(t % NR, mid)

        gather(0, 0)
        gather(1, 1)

        pltpu.sync_copy(h_hbm.at[pl.ds(s * RPT, RPT)], acc.at[pl.ds(s * RPT, RPT)])

        @pl.when(s == NS - 1)
        def _():
            pltpu.sync_copy(h_hbm.at[pl.ds(RPT * NS, TAIL)],
                            acc.at[pl.ds(RPT * NS, TAIL)])

        plsc.subcore_barrier()

        def body3(i, carry):
            j0 = NR * i
            for t in range(NR):
                slot(j0 + t, t, first=False)
            return carry

        M = (NCHUNK - NR - 2) // NR
        for t in range(NR):
            slot(t, t, first=(t == 0))
        lax.fori_loop(1, 1 + M, body3, 0)
        for j in range(NR * (1 + M), NCHUNK):
            slot(j, j % NR, g_ok=(j + 2 < NCHUNK))
        wait_scatter((NCHUNK - 1) % NR)

        plsc.subcore_barrier()
        pltpu.sync_copy(acc.at[pl.ds(s * RPT, RPT)],
                        out_hbm.at[c, pl.ds(s * RPT, RPT)])

        @pl.when(s == NS - 1)
        def _():
            pltpu.sync_copy(acc.at[pl.ds(RPT * NS, TAIL)],
                            out_hbm.at[c, pl.ds(RPT * NS, TAIL)])

    return seg(h, packed)


def _tc_tail(h, p0, p1, snorm_n, W1, b1, W2, b2, gamma, beta):
    def body(h_ref, p0_ref, p1_ref, sn_ref, w1_ref, b1_ref, w2_ref, b2_ref,
             g_ref, be_ref, o_ref):
        hh = h_ref[...]
        x = p0_ref[...] + p1_ref[...] - hh
        x = jnp.maximum(
            jnp.dot(x, w1_ref[...], preferred_element_type=jnp.float32)
            + b1_ref[...], 0.0)
        x = jnp.dot(x, w2_ref[...], preferred_element_type=jnp.float32) + b2_ref[...]
        x = x * sn_ref[...]
        mean = jnp.mean(x, axis=0, keepdims=True)
        xc = x - mean
        var = jnp.mean(xc * xc, axis=0, keepdims=True)
        y = xc * lax.rsqrt(var + BN_EPS) * g_ref[...] + be_ref[...]
        o_ref[...] = hh + jnp.maximum(y, 0.0)

    return pl.pallas_call(
        body,
        out_shape=jax.ShapeDtypeStruct((N, D), jnp.float32),
    )(h, p0, p1, snorm_n, W1, b1, W2, b2, gamma, beta)


def kernel(h, edge_index, edge_mask, snorm_n, W1, b1, W2, b2, gamma, beta):
    src = edge_index[0].reshape(E // K, K)
    dst = edge_index[1].reshape(E // K, K)
    mbits = lax.bitcast_convert_type(edge_mask[:, 0], jnp.int32).reshape(E // K, K)
    packed = jnp.stack([src, dst, mbits], axis=1)
    part = _sc_segment(h, packed)
    return _tc_tail(h, part[0], part[1], snorm_n, W1, b1, W2, b2, gamma, beta)


# speedup vs baseline: 1.1612x; 1.0925x over previous
import functools

import jax
import jax.numpy as jnp
from jax import lax
from jax.experimental import pallas as pl
from jax.experimental.pallas import tpu as pltpu
from jax.experimental.pallas import tpu_sc as plsc

N = 10000
D = 128
E = 320000
BN_EPS = 1e-5

NC, NS, L = 2, 16, 16
NW = NC * NS
EPW = E // NW
K = 80
NCHUNK = EPW // K
RPT = 624
TAIL = N - RPT * NS


def _sc_segment(h, packed):
    mesh = plsc.VectorSubcoreMesh(core_axis_name="c", subcore_axis_name="s")

    NR = 4

    @functools.partial(
        pl.kernel,
        out_type=jax.ShapeDtypeStruct((NC, N, D), jnp.float32),
        mesh=mesh,
        scratch_types=(
            [pltpu.VMEM((K, D), jnp.float32) for _ in range(NR)]
            + [pltpu.VMEM((3, K), jnp.int32) for _ in range(NR)]
            + [pltpu.VMEM_SHARED((N, D), jnp.float32)]
            + [pltpu.SemaphoreType.DMA for _ in range(2 * NR)]
        ),
    )
    def seg(h_hbm, pk_hbm, out_hbm, *scr):
        rows = scr[:NR]
        desc = scr[NR:2 * NR]
        acc = scr[2 * NR]
        gsem = scr[2 * NR + 1:3 * NR + 1]
        ssem = scr[3 * NR + 1:]
        c = lax.axis_index("c")
        s = lax.axis_index("s")
        wid = s * NC + c
        base = wid * NCHUNK

        def gather(j, b):
            pltpu.sync_copy(pk_hbm.at[base + j], desc[b])
            pltpu.async_copy(h_hbm.at[desc[b].at[0]], rows[b], gsem[b])

        def wait_scatter(b):
            pltpu.make_async_copy(rows[b], acc.at[desc[b].at[1]],
                                  ssem[b]).wait()

        def process(b):
            pltpu.make_async_copy(h_hbm.at[desc[b].at[0]], rows[b],
                                  gsem[b]).wait()

            def scale(t, carry2):
                m16 = desc[b][2, pl.ds(t * L, L)]
                for e in range(L):
                    m = lax.bitcast_convert_type(m16[e], jnp.float32)
                    r = t * L + e
                    for g in range(D // L):
                        rows[b][r, pl.ds(g * L, L)] = (
                            rows[b][r, pl.ds(g * L, L)] * m)
                return carry2

            lax.fori_loop(0, K // L, scale, 0)
            pltpu.async_copy(rows[b], acc.at[desc[b].at[1]], ssem[b],
                             add=True)

        def slot(j, t, first=False, g_ok=True):
            process(t % NR)
            if not first:
                wait_scatter((t - 2) % NR)
            if g_ok:
                gather(j + 2, (t + 2) % NR)

        gather(0, 0)
        gather(1, 1)

        pltpu.sync_copy(h_hbm.at[pl.ds(s * RPT, RPT)], acc.at[pl.ds(s * RPT, RPT)])

        @pl.when(s == NS - 1)
        def _():
            pltpu.sync_copy(h_hbm.at[pl.ds(RPT * NS, TAIL)],
                            acc.at[pl.ds(RPT * NS, TAIL)])

        plsc.subcore_barrier()

        def body3(i, carry):
            j0 = NR * i
            for t in range(NR):
                slot(j0 + t, t, first=False)
            return carry

        M = (NCHUNK - NR - 2) // NR
        for t in range(NR):
            slot(t, t, first=(t < 2))
        lax.fori_loop(1, 1 + M, body3, 0)
        for j in range(NR * (1 + M), NCHUNK):
            slot(j, j % NR, g_ok=(j + 2 < NCHUNK))
        wait_scatter((NCHUNK - 2) % NR)
        wait_scatter((NCHUNK - 1) % NR)

        plsc.subcore_barrier()
        pltpu.sync_copy(acc.at[pl.ds(s * RPT, RPT)],
                        out_hbm.at[c, pl.ds(s * RPT, RPT)])

        @pl.when(s == NS - 1)
        def _():
            pltpu.sync_copy(acc.at[pl.ds(RPT * NS, TAIL)],
                            out_hbm.at[c, pl.ds(RPT * NS, TAIL)])

    return seg(h, packed)


def _tc_tail(h, p0, p1, snorm_n, W1, b1, W2, b2, gamma, beta):
    def body(h_ref, p0_ref, p1_ref, sn_ref, w1_ref, b1_ref, w2_ref, b2_ref,
             g_ref, be_ref, o_ref):
        hh = h_ref[...]
        x = p0_ref[...] + p1_ref[...] - hh
        x = jnp.maximum(
            jnp.dot(x, w1_ref[...], preferred_element_type=jnp.float32)
            + b1_ref[...], 0.0)
        x = jnp.dot(x, w2_ref[...], preferred_element_type=jnp.float32) + b2_ref[...]
        x = x * sn_ref[...]
        mean = jnp.mean(x, axis=0, keepdims=True)
        xc = x - mean
        var = jnp.mean(xc * xc, axis=0, keepdims=True)
        y = xc * lax.rsqrt(var + BN_EPS) * g_ref[...] + be_ref[...]
        o_ref[...] = hh + jnp.maximum(y, 0.0)

    return pl.pallas_call(
        body,
        out_shape=jax.ShapeDtypeStruct((N, D), jnp.float32),
    )(h, p0, p1, snorm_n, W1, b1, W2, b2, gamma, beta)


def kernel(h, edge_index, edge_mask, snorm_n, W1, b1, W2, b2, gamma, beta):
    src = edge_index[0].reshape(E // K, K)
    dst = edge_index[1].reshape(E // K, K)
    mbits = lax.bitcast_convert_type(edge_mask[:, 0], jnp.int32).reshape(E // K, K)
    packed = jnp.stack([src, dst, mbits], axis=1)
    part = _sc_segment(h, packed)
    return _tc_tail(h, part[0], part[1], snorm_n, W1, b1, W2, b2, gamma, beta)


# speedup vs baseline: 1.1970x; 1.0308x over previous
import functools

import jax
import jax.numpy as jnp
from jax import lax
from jax.experimental import pallas as pl
from jax.experimental.pallas import tpu as pltpu
from jax.experimental.pallas import tpu_sc as plsc

N = 10000
D = 128
E = 320000
BN_EPS = 1e-5

NC, NS, L = 2, 16, 16
NW = NC * NS
EPW = E // NW
K = 80
NCHUNK = EPW // K
RPT = 624
TAIL = N - RPT * NS


def _sc_segment(h, packed):
    mesh = plsc.VectorSubcoreMesh(core_axis_name="c", subcore_axis_name="s")

    NR = 4

    @functools.partial(
        pl.kernel,
        out_type=jax.ShapeDtypeStruct((NC, N, D), jnp.float32),
        mesh=mesh,
        scratch_types=(
            [pltpu.VMEM((K, D), jnp.float32) for _ in range(NR)]
            + [pltpu.VMEM((3, K), jnp.int32) for _ in range(NR)]
            + [pltpu.VMEM_SHARED((N, D), jnp.float32)]
            + [pltpu.SemaphoreType.DMA for _ in range(2 * NR)]
        ),
    )
    def seg(h_hbm, pk_hbm, out_hbm, *scr):
        rows = scr[:NR]
        desc = scr[NR:2 * NR]
        acc = scr[2 * NR]
        gsem = scr[2 * NR + 1:3 * NR + 1]
        ssem = scr[3 * NR + 1:]
        c = lax.axis_index("c")
        s = lax.axis_index("s")
        wid = s * NC + c
        base = wid * NCHUNK

        def gather(j, b):
            pltpu.sync_copy(pk_hbm.at[base + j], desc[b])
            pltpu.async_copy(h_hbm.at[desc[b].at[0]], rows[b], gsem[b])

        def wait_scatter(b):
            pltpu.make_async_copy(rows[b], acc.at[desc[b].at[1]],
                                  ssem[b]).wait()

        def wait_gather(b):
            pltpu.make_async_copy(h_hbm.at[desc[b].at[0]], rows[b],
                                  gsem[b]).wait()

        def process(b):
            def scale(t, carry2):
                m16 = desc[b][2, pl.ds(t * L, L)]
                for e in range(L):
                    m = lax.bitcast_convert_type(m16[e], jnp.float32)
                    r = t * L + e
                    for g in range(D // L):
                        rows[b][r, pl.ds(g * L, L)] = (
                            rows[b][r, pl.ds(g * L, L)] * m)
                return carry2

            lax.fori_loop(0, K // L, scale, 0)
            pltpu.async_copy(rows[b], acc.at[desc[b].at[1]], ssem[b],
                             add=True)

        def slot(j, t, first=False, g_ok=True):
            wait_gather(t % NR)
            if not first:
                wait_scatter((t - 2) % NR)
            if g_ok:
                gather(j + 2, (t + 2) % NR)
            process(t % NR)

        gather(0, 0)
        gather(1, 1)

        pltpu.sync_copy(h_hbm.at[pl.ds(s * RPT, RPT)], acc.at[pl.ds(s * RPT, RPT)])

        @pl.when(s == NS - 1)
        def _():
            pltpu.sync_copy(h_hbm.at[pl.ds(RPT * NS, TAIL)],
                            acc.at[pl.ds(RPT * NS, TAIL)])

        plsc.subcore_barrier()

        def body3(i, carry):
            j0 = NR * i
            for t in range(NR):
                slot(j0 + t, t, first=False)
            return carry

        M = (NCHUNK - NR - 2) // NR
        for t in range(NR):
            slot(t, t, first=(t < 2))
        lax.fori_loop(1, 1 + M, body3, 0)
        for j in range(NR * (1 + M), NCHUNK):
            slot(j, j % NR, g_ok=(j + 2 < NCHUNK))
        wait_scatter((NCHUNK - 2) % NR)
        wait_scatter((NCHUNK - 1) % NR)

        plsc.subcore_barrier()
        pltpu.sync_copy(acc.at[pl.ds(s * RPT, RPT)],
                        out_hbm.at[c, pl.ds(s * RPT, RPT)])

        @pl.when(s == NS - 1)
        def _():
            pltpu.sync_copy(acc.at[pl.ds(RPT * NS, TAIL)],
                            out_hbm.at[c, pl.ds(RPT * NS, TAIL)])

    return seg(h, packed)


def _tc_tail(h, p0, p1, snorm_n, W1, b1, W2, b2, gamma, beta):
    def body(h_ref, p0_ref, p1_ref, sn_ref, w1_ref, b1_ref, w2_ref, b2_ref,
             g_ref, be_ref, o_ref):
        hh = h_ref[...]
        x = p0_ref[...] + p1_ref[...] - hh
        x = jnp.maximum(
            jnp.dot(x, w1_ref[...], preferred_element_type=jnp.float32)
            + b1_ref[...], 0.0)
        x = jnp.dot(x, w2_ref[...], preferred_element_type=jnp.float32) + b2_ref[...]
        x = x * sn_ref[...]
        mean = jnp.mean(x, axis=0, keepdims=True)
        xc = x - mean
        var = jnp.mean(xc * xc, axis=0, keepdims=True)
        y = xc * lax.rsqrt(var + BN_EPS) * g_ref[...] + be_ref[...]
        o_ref[...] = hh + jnp.maximum(y, 0.0)

    return pl.pallas_call(
        body,
        out_shape=jax.ShapeDtypeStruct((N, D), jnp.float32),
    )(h, p0, p1, snorm_n, W1, b1, W2, b2, gamma, beta)


def kernel(h, edge_index, edge_mask, snorm_n, W1, b1, W2, b2, gamma, beta):
    src = edge_index[0].reshape(E // K, K)
    dst = edge_index[1].reshape(E // K, K)
    mbits = lax.bitcast_convert_type(edge_mask[:, 0], jnp.int32).reshape(E // K, K)
    packed = jnp.stack([src, dst, mbits], axis=1)
    part = _sc_segment(h, packed)
    return _tc_tail(h, part[0], part[1], snorm_n, W1, b1, W2, b2, gamma, beta)


# speedup vs baseline: 1.2353x; 1.0319x over previous
import functools

import jax
import jax.numpy as jnp
from jax import lax
from jax.experimental import pallas as pl
from jax.experimental.pallas import tpu as pltpu
from jax.experimental.pallas import tpu_sc as plsc

N = 10000
D = 128
E = 320000
BN_EPS = 1e-5

NC, NS, L = 2, 16, 16
NW = NC * NS
EPW = E // NW
K = 80
NCHUNK = EPW // K
RPT = 624
TAIL = N - RPT * NS


def _sc_segment(h, packed):
    mesh = plsc.VectorSubcoreMesh(core_axis_name="c", subcore_axis_name="s")

    NR = 4
    NQ = 3
    UNROLL = 12

    @functools.partial(
        pl.kernel,
        out_type=jax.ShapeDtypeStruct((NC, N, D), jnp.float32),
        mesh=mesh,
        scratch_types=(
            [pltpu.VMEM((K, D), jnp.float32) for _ in range(NR)]
            + [pltpu.VMEM((4, 3, K), jnp.int32) for _ in range(NQ)]
            + [pltpu.VMEM_SHARED((N, D), jnp.float32)]
            + [pltpu.SemaphoreType.DMA for _ in range(2 * NR)]
        ),
    )
    def seg(h_hbm, pk_hbm, out_hbm, *scr):
        rows = scr[:NR]
        desc = scr[NR:NR + NQ]
        acc = scr[NR + NQ]
        gsem = scr[NR + NQ + 1:NR + NQ + 1 + NR]
        ssem = scr[NR + NQ + 1 + NR:]
        c = lax.axis_index("c")
        s = lax.axis_index("s")
        wid = s * NC + c
        base = wid * NCHUNK

        def pos(t):
            return t % NR, (t // 4) % NQ, t % 4

        def gather(t):
            b, q, u = pos(t)
            pltpu.async_copy(h_hbm.at[desc[q].at[u, 0]], rows[b], gsem[b])

        def wait_gather(t):
            b, q, u = pos(t)
            pltpu.make_async_copy(h_hbm.at[desc[q].at[u, 0]], rows[b],
                                  gsem[b]).wait()

        def wait_scatter(t):
            b, q, u = pos(t)
            pltpu.make_async_copy(rows[b], acc.at[desc[q].at[u, 1]],
                                  ssem[b]).wait()

        def process(t):
            b, q, u = pos(t)

            def scale(tt, carry2):
                m16 = desc[q][u, 2, pl.ds(tt * L, L)]
                for e in range(L):
                    m = lax.bitcast_convert_type(m16[e], jnp.float32)
                    r = tt * L + e
                    for g in range(D // L):
                        rows[b][r, pl.ds(g * L, L)] = (
                            rows[b][r, pl.ds(g * L, L)] * m)
                return carry2

            lax.fori_loop(0, K // L, scale, 0)
            pltpu.async_copy(rows[b], acc.at[desc[q].at[u, 1]], ssem[b],
                             add=True)

        def slot(j, t, guard_first=False, g_ok=True):
            wait_gather(t)
            if guard_first:
                @pl.when(j >= 2)
                def _():
                    wait_scatter(t - 2)
            else:
                wait_scatter(t - 2)
            if t % 4 == 1:
                pltpu.sync_copy(pk_hbm.at[pl.ds(base + j + 3, 4)],
                                desc[(t // 4 + 1) % NQ])
            if g_ok:
                gather(t + 2)
            process(t)

        pltpu.sync_copy(pk_hbm.at[pl.ds(base, 4)], desc[0])
        gather(0)
        gather(1)

        pltpu.sync_copy(h_hbm.at[pl.ds(s * RPT, RPT)], acc.at[pl.ds(s * RPT, RPT)])

        @pl.when(s == NS - 1)
        def _():
            pltpu.sync_copy(h_hbm.at[pl.ds(RPT * NS, TAIL)],
                            acc.at[pl.ds(RPT * NS, TAIL)])

        plsc.subcore_barrier()

        def body12(i, carry):
            j0 = UNROLL * i
            for t in range(UNROLL):
                slot(j0 + t, t, guard_first=(t < 2))
            return carry

        M = (NCHUNK - UNROLL - 2) // UNROLL
        lax.fori_loop(0, 1 + M, body12, 0)
        for j in range(UNROLL * (1 + M), NCHUNK):
            slot(j, j % UNROLL, g_ok=(j + 2 < NCHUNK))
        wait_scatter((NCHUNK - 2) % UNROLL)
        wait_scatter((NCHUNK - 1) % UNROLL)

        plsc.subcore_barrier()
        pltpu.sync_copy(acc.at[pl.ds(s * RPT, RPT)],
                        out_hbm.at[c, pl.ds(s * RPT, RPT)])

        @pl.when(s == NS - 1)
        def _():
            pltpu.sync_copy(acc.at[pl.ds(RPT * NS, TAIL)],
                            out_hbm.at[c, pl.ds(RPT * NS, TAIL)])

    return seg(h, packed)


def _tc_tail(h, p0, p1, snorm_n, W1, b1, W2, b2, gamma, beta):
    def body(h_ref, p0_ref, p1_ref, sn_ref, w1_ref, b1_ref, w2_ref, b2_ref,
             g_ref, be_ref, o_ref):
        hh = h_ref[...]
        x = p0_ref[...] + p1_ref[...] - hh
        x = jnp.maximum(
            jnp.dot(x, w1_ref[...], preferred_element_type=jnp.float32)
            + b1_ref[...], 0.0)
        x = jnp.dot(x, w2_ref[...], preferred_element_type=jnp.float32) + b2_ref[...]
        x = x * sn_ref[...]
        mean = jnp.mean(x, axis=0, keepdims=True)
        xc = x - mean
        var = jnp.mean(xc * xc, axis=0, keepdims=True)
        y = xc * lax.rsqrt(var + BN_EPS) * g_ref[...] + be_ref[...]
        o_ref[...] = hh + jnp.maximum(y, 0.0)

    return pl.pallas_call(
        body,
        out_shape=jax.ShapeDtypeStruct((N, D), jnp.float32),
    )(h, p0, p1, snorm_n, W1, b1, W2, b2, gamma, beta)


def kernel(h, edge_index, edge_mask, snorm_n, W1, b1, W2, b2, gamma, beta):
    src = edge_index[0].reshape(E // K, K)
    dst = edge_index[1].reshape(E // K, K)
    mbits = lax.bitcast_convert_type(edge_mask[:, 0], jnp.int32).reshape(E // K, K)
    packed = jnp.stack([src, dst, mbits], axis=1)
    packed = jnp.concatenate(
        [packed, jnp.zeros((3, 3, K), jnp.int32)], axis=0)
    part = _sc_segment(h, packed)
    return _tc_tail(h, part[0], part[1], snorm_n, W1, b1, W2, b2, gamma, beta)
